# baseline (speedup 1.0000x reference)
import functools

import jax
import jax.numpy as jnp
import numpy as np
from jax.experimental import pallas as pl
from jax.experimental.pallas import tpu as pltpu
from jax.experimental.pallas import tpu_sc as plsc

S, H = 2048, 1024
NH, NKV, HD = 16, 8, 64
I, E = 2048, 8
EPS = 1e-6
EP = 16
NA = 2 * S
BT = 256
NBE = NA // BT + E
NBS = S // BT
NBLK = NBE + NBS
NPADE = NBE * BT
NPAD = NBLK * BT
HW = H // 2


def _rms_mm_kernel(x_ref, ln_ref, w_ref, o_ref):
    x = x_ref[...]
    h = x * jax.lax.rsqrt(jnp.mean(x * x, axis=-1, keepdims=True) + EPS)
    h = h * ln_ref[...]
    o_ref[...] = jnp.dot(h.astype(jnp.bfloat16), w_ref[...],
                         preferred_element_type=jnp.float32).astype(jnp.bfloat16)


BQ = 512
BK = 512
NC = S // BQ


def _flash_kernel(q_ref, k_ref, v_ref, wo_ref, x_ref, o_ref,
                  m_scr, l_scr, acc_scr, oall_scr, mask_scr):
    ci = pl.program_id(0)
    hp = pl.program_id(1)
    cj = pl.program_id(2)
    scale = 1.0 / np.sqrt(HD)
    B2 = 2 * BQ

    @pl.when((ci == 0) & (hp == 0) & (cj == 0))
    def _():
        rl = jax.lax.broadcasted_iota(jnp.int32, (B2, BK), 0) % BQ
        cl = jax.lax.broadcasted_iota(jnp.int32, (B2, BK), 1)
        mask_scr[...] = jnp.where(cl <= rl, 0.0, -1e30)

    @pl.when(cj == 0)
    def _():
        m_scr[...] = jnp.full((B2, 128), -1e30, jnp.float32)
        l_scr[...] = jnp.zeros((B2, 128), jnp.float32)
        acc_scr[...] = jnp.zeros((B2, HD), jnp.float32)

    @pl.when(cj <= ci)
    def _():
        q2 = jnp.concatenate([q_ref[0], q_ref[1]], axis=0)
        s = jax.lax.dot_general(q2, k_ref[0], (((1,), (1,)), ((), ())),
                                preferred_element_type=jnp.float32) * scale
        s = jax.lax.cond(cj == ci, lambda: s + mask_scr[...], lambda: s)
        m_old = m_scr[:, 0:1]
        m_new = jnp.maximum(m_old, jnp.max(s, axis=-1, keepdims=True))
        p = jnp.exp(s - m_new)
        corr = jnp.exp(m_old - m_new)
        l_new = l_scr[:, 0:1] * corr + jnp.sum(p, axis=-1, keepdims=True)
        m_scr[...] = jnp.broadcast_to(m_new, (B2, 128))
        l_scr[...] = jnp.broadcast_to(l_new, (B2, 128))
        pv = jax.lax.dot_general(p.astype(jnp.bfloat16), v_ref[0],
                                 (((1,), (0,)), ((), ())),
                                 preferred_element_type=jnp.float32)
        acc_scr[...] = acc_scr[...] * corr + pv

    @pl.when(cj == ci)
    def _():
        o_h = acc_scr[...] / l_scr[:, 0:1]
        o_pair = jnp.concatenate([o_h[:BQ], o_h[BQ:]], axis=1)
        oall_scr[:, pl.ds(hp * 2 * HD, 2 * HD)] = o_pair.astype(jnp.bfloat16)

    @pl.when((hp == NKV - 1) & (cj == ci))
    def _():
        o_ref[...] = x_ref[...] + jax.lax.dot_general(
            oall_scr[...], wo_ref[...], (((1,), (0,)), ((), ())),
            preferred_element_type=jnp.float32)


def _router_kernel(x_ref, ln_ref, gw_ref, h_ref, p_ref, v1_ref, v2_ref,
                   meta_ref, aux_ref):
    x = x_ref[...]
    h = x * jax.lax.rsqrt(jnp.mean(x * x, axis=-1, keepdims=True) + EPS)
    h = h * ln_ref[...]
    h_ref[...] = h.astype(jnp.bfloat16)
    logits = jax.lax.dot_general(h, gw_ref[...], (((1,), (0,)), ((), ())),
                                 precision=jax.lax.Precision.HIGHEST,
                                 preferred_element_type=jnp.float32)
    lane = jax.lax.broadcasted_iota(jnp.int32, (S, EP), 1)
    logits = jnp.where(lane < E, logits, -jnp.inf)
    lm = jnp.max(logits, axis=-1, keepdims=True)
    ex = jnp.exp(logits - lm)
    probs = ex / jnp.sum(ex, axis=-1, keepdims=True)
    m1 = jnp.max(probs, axis=-1, keepdims=True)
    i1 = jnp.min(jnp.where(probs == m1, lane, EP), axis=-1, keepdims=True)
    mask1 = lane == i1
    pr2 = jnp.where(mask1, -jnp.inf, probs)
    m2 = jnp.max(pr2, axis=-1, keepdims=True)
    i2 = jnp.min(jnp.where(pr2 == m2, lane, EP), axis=-1, keepdims=True)
    mask2 = lane == i2
    v1_ref[...] = m1
    v2_ref[...] = m2
    counts = jnp.sum(jnp.where(mask1 | mask2, 1.0, 0.0), axis=0, keepdims=True)
    importance = jnp.mean(probs, axis=0, keepdims=True)
    aux = jnp.sum(importance * counts) * (E / (NA * 1.0))
    aux_ref[...] = jnp.full((1, 128), aux, jnp.float32)
    O = jnp.concatenate([jnp.where(mask1, 1.0, 0.0),
                         jnp.where(mask2, 1.0, 0.0)], axis=0)
    CB = 512
    rl = jax.lax.broadcasted_iota(jnp.int32, (CB, CB), 0)
    cl = jax.lax.broadcasted_iota(jnp.int32, (CB, CB), 1)
    ltri = jnp.where(cl < rl, 1.0, 0.0)
    carry = jnp.zeros((1, EP), jnp.float32)
    ranks = []
    for i in range(NA // CB):
        ob = O[i * CB:(i + 1) * CB]
        cb = jax.lax.dot_general(ltri, ob, (((1,), (0,)), ((), ())),
                                 preferred_element_type=jnp.float32) + carry
        ranks.append(jnp.sum(cb * ob, axis=-1, keepdims=True))
        carry = carry + jnp.sum(ob, axis=0, keepdims=True)
    rank = jnp.concatenate(ranks, axis=0)
    nb_row = jnp.floor((carry + (BT - 1.0)) * (1.0 / BT))
    er = jax.lax.broadcasted_iota(jnp.int32, (EP, EP), 0)
    ec = jax.lax.broadcasted_iota(jnp.int32, (EP, EP), 1)
    ustrict = jnp.where(er < ec, 1.0, 0.0)
    pad_off = jax.lax.dot_general(nb_row, ustrict, (((1,), (0,)), ((), ())),
                                  preferred_element_type=jnp.float32) * BT
    off = jnp.sum(O * pad_off, axis=-1, keepdims=True)
    p_ref[...] = (rank + off).astype(jnp.int32)
    ones_col = jnp.ones((NA, 1), jnp.float32)
    counts_col = jax.lax.dot_general(O, ones_col, (((0,), (0,)), ((), ())),
                                     preferred_element_type=jnp.float32)
    nb_col = jnp.floor((counts_col + (BT - 1.0)) * (1.0 / BT))
    lincl = jnp.where(er >= ec, 1.0, 0.0)
    cum_col = jax.lax.dot_general(lincl, nb_col, (((1,), (0,)), ((), ())),
                                  preferred_element_type=jnp.float32)
    e_col = jax.lax.broadcasted_iota(jnp.int32, (EP, 1), 0)
    b_iota = jax.lax.broadcasted_iota(jnp.int32, (1, 64), 1)
    mat = jnp.where((b_iota >= cum_col.astype(jnp.int32)) & (e_col < E),
                    1.0, 0.0)
    eob = jnp.sum(mat, axis=0, keepdims=True)
    nused = jnp.sum(jnp.where(e_col < E, nb_col, 0.0))
    meta_ref[...] = jnp.where(b_iota == 32, nused, eob).astype(jnp.int32)


NW = 32


def _make_sc_dispatch(mesh):
    @functools.partial(
        pl.kernel, mesh=mesh,
        out_type=jax.ShapeDtypeStruct((NPAD,), jnp.int32),
        scratch_types=[
            pltpu.VMEM((NA // NW,), jnp.int32),
            pltpu.VMEM((NA // NW,), jnp.int32),
            pltpu.VMEM((S // NW,), jnp.int32),
            pltpu.SemaphoreType.DMA,
        ])
    def _sc_dispatch(p_hbm, out_hbm, idx_v, val_v, sh_v, sem):
        wid = jax.lax.axis_index("s") * 2 + jax.lax.axis_index("c")
        na_w = NA // NW
        base = wid * na_w
        pltpu.sync_copy(p_hbm.at[pl.ds(base, na_w)], idx_v)
        lane = jax.lax.broadcasted_iota(jnp.int32, (16,), 0)
        for i in range(na_w // 16):
            v = base + i * 16 + lane
            val_v[pl.ds(i * 16, 16)] = jnp.where(v < S, v, v - S)
        pltpu.async_copy(val_v, out_hbm.at[idx_v], sem).wait()
        s_w = S // NW
        sbase = wid * s_w
        for i in range(s_w // 16):
            sh_v[pl.ds(i * 16, 16)] = sbase + i * 16 + lane
        pltpu.sync_copy(sh_v, out_hbm.at[pl.ds(NPADE + sbase, s_w)])

    return _sc_dispatch


def _make_sc_gather(mesh, V, B):
    nper = B // NW
    CH = min(nper, 128)
    nch = nper // CH

    @functools.partial(
        pl.kernel, mesh=mesh,
        out_type=jax.ShapeDtypeStruct((B, HW), jnp.int32),
        scratch_types=[
            pltpu.VMEM((nper,), jnp.int32),
            pltpu.VMEM((CH, HW), jnp.int32),
            pltpu.SemaphoreType.DMA,
        ])
    def _g(tab_hbm, idx_hbm, out_hbm, idx_v, rows_v, sem):
        wid = jax.lax.axis_index("s") * 2 + jax.lax.axis_index("c")
        base = wid * nper
        pltpu.sync_copy(idx_hbm.at[pl.ds(base, nper)], idx_v)
        for i in range(nper // 16):
            x = idx_v[pl.ds(i * 16, 16)]
            idx_v[pl.ds(i * 16, 16)] = jnp.clip(x, 0, V - 1)
        for ci in range(nch):
            pltpu.async_copy(tab_hbm.at[idx_v.at[pl.ds(ci * CH, CH)]],
                             rows_v, sem).wait()
            pltpu.sync_copy(rows_v, out_hbm.at[pl.ds(base + ci * CH, CH)])

    return _g


@functools.cache
def _get_sc_fns():
    mesh = plsc.VectorSubcoreMesh(core_axis_name="c", subcore_axis_name="s")
    return (_make_sc_dispatch(mesh),
            _make_sc_gather(mesh, S, NPAD),
            _make_sc_gather(mesh, NPAD, NA))


def _gmm_kernel(eob_ref, nused_ref, xs_ref, w1_ref, w2_ref, o_ref):
    b = pl.program_id(0)

    @pl.when((b < nused_ref[0]) | (b >= NBE))
    def _():
        h1 = jax.lax.dot_general(xs_ref[...], w1_ref[0],
                                 (((1,), (1,)), ((), ())),
                                 preferred_element_type=jnp.float32)
        h1 = (h1 * jax.nn.sigmoid(h1)).astype(jnp.bfloat16)
        o_ref[...] = jax.lax.dot_general(
            h1, w2_ref[0], (((1,), (1,)), ((), ())),
            preferred_element_type=jnp.float32).astype(jnp.bfloat16)


def _combine_kernel(x_ref, es_ref, g1_ref, g2_ref, v1_ref, v2_ref, o_ref):
    o_ref[...] = (x_ref[...] + es_ref[...].astype(jnp.float32)
                  + v1_ref[...] * g1_ref[...].astype(jnp.float32)
                  + v2_ref[...] * g2_ref[...].astype(jnp.float32))


def kernel(x, wq, wk, wv, wo, gate_w, w1, w2, sw1, sw2, ln1, ln2):
    xf = x[0]
    wqkv_t = jnp.concatenate([wq, wk, wv], axis=0).T.astype(jnp.bfloat16)
    wo_t = wo.T.astype(jnp.bfloat16)
    gw_t = jnp.pad(gate_w, ((0, EP - E), (0, 0))).T
    w1s = jnp.concatenate([w1, sw1[None]], axis=0).astype(jnp.bfloat16)
    w2s = jnp.concatenate([w2, sw2[None]], axis=0).astype(jnp.bfloat16)

    qkv = pl.pallas_call(
        _rms_mm_kernel,
        out_shape=jax.ShapeDtypeStruct((S, NH * HD + 2 * NKV * HD),
                                       jnp.bfloat16),
    )(xf, ln1.reshape(1, H), wqkv_t)

    q3 = qkv[:, :NH * HD].reshape(S, NH, HD).transpose(1, 0, 2)
    k3 = qkv[:, NH * HD:(NH + NKV) * HD].reshape(S, NKV, HD).transpose(1, 0, 2)
    v3 = qkv[:, (NH + NKV) * HD:].reshape(S, NKV, HD).transpose(1, 0, 2)

    x2 = pl.pallas_call(
        _flash_kernel,
        grid=(NC, NKV, S // BK),
        in_specs=[
            pl.BlockSpec((2, BQ, HD), lambda ci, hp, cj: (hp, ci, 0)),
            pl.BlockSpec((1, BK, HD), lambda ci, hp, cj: (hp, cj, 0)),
            pl.BlockSpec((1, BK, HD), lambda ci, hp, cj: (hp, cj, 0)),
            pl.BlockSpec((NH * HD, H), lambda ci, hp, cj: (0, 0)),
            pl.BlockSpec((BQ, H), lambda ci, hp, cj: (ci, 0)),
        ],
        out_specs=pl.BlockSpec((BQ, H), lambda ci, hp, cj: (ci, 0)),
        out_shape=jax.ShapeDtypeStruct((S, H), jnp.float32),
        scratch_shapes=[
            pltpu.VMEM((2 * BQ, 128), jnp.float32),
            pltpu.VMEM((2 * BQ, 128), jnp.float32),
            pltpu.VMEM((2 * BQ, HD), jnp.float32),
            pltpu.VMEM((BQ, NH * HD), jnp.bfloat16),
            pltpu.VMEM((2 * BQ, BK), jnp.float32),
        ],
        compiler_params=pltpu.CompilerParams(
            dimension_semantics=("arbitrary", "arbitrary", "arbitrary")),
    )(q3, k3, v3, wo_t, xf)

    h2, p_pos, v1, v2, meta, aux = pl.pallas_call(
        _router_kernel,
        out_shape=[
            jax.ShapeDtypeStruct((S, H), jnp.bfloat16),
            jax.ShapeDtypeStruct((NA, 1), jnp.int32),
            jax.ShapeDtypeStruct((S, 1), jnp.float32),
            jax.ShapeDtypeStruct((S, 1), jnp.float32),
            jax.ShapeDtypeStruct((1, 64), jnp.int32),
            jax.ShapeDtypeStruct((1, 128), jnp.float32),
        ],
    )(x2, ln2.reshape(1, H), gw_t)

    sc_dispatch, sc_gather_xs, sc_gather_eo = _get_sc_fns()
    p_flat = p_pos.reshape(NA)
    tok_sorted = sc_dispatch(p_flat)

    h32 = jax.lax.bitcast_convert_type(h2.reshape(S, HW, 2), jnp.int32)
    xs32 = sc_gather_xs(h32, tok_sorted)
    xs = jax.lax.bitcast_convert_type(xs32, jnp.bfloat16).reshape(NPAD, H)

    eob = meta[0, :32]
    nused = meta[0, 32:33]
    eo = pl.pallas_call(
        _gmm_kernel,
        grid_spec=pltpu.PrefetchScalarGridSpec(
            num_scalar_prefetch=2,
            grid=(NBLK,),
            in_specs=[
                pl.BlockSpec((BT, H), lambda b, eob, nu: (b, 0)),
                pl.BlockSpec((1, I, H), lambda b, eob, nu: (eob[b], 0, 0)),
                pl.BlockSpec((1, H, I), lambda b, eob, nu: (eob[b], 0, 0)),
            ],
            out_specs=pl.BlockSpec((BT, H), lambda b, eob, nu: (b, 0)),
        ),
        out_shape=jax.ShapeDtypeStruct((NPAD, H), jnp.bfloat16),
        compiler_params=pltpu.CompilerParams(
            dimension_semantics=("arbitrary",)),
    )(eob, nused, xs, w1s, w2s)

    eo32 = jax.lax.bitcast_convert_type(eo.reshape(NPAD, HW, 2), jnp.int32)
    g32 = sc_gather_eo(eo32, p_flat)
    g = jax.lax.bitcast_convert_type(g32, jnp.bfloat16).reshape(NA, H)

    y = pl.pallas_call(
        _combine_kernel,
        grid=(NC,),
        in_specs=[
            pl.BlockSpec((BQ, H), lambda i: (i, 0)),
            pl.BlockSpec((BQ, H), lambda i: (i, 0)),
            pl.BlockSpec((BQ, H), lambda i: (i, 0)),
            pl.BlockSpec((BQ, H), lambda i: (i, 0)),
            pl.BlockSpec((BQ, 1), lambda i: (i, 0)),
            pl.BlockSpec((BQ, 1), lambda i: (i, 0)),
        ],
        out_specs=pl.BlockSpec((BQ, H), lambda i: (i, 0)),
        out_shape=jax.ShapeDtypeStruct((S, H), jnp.float32),
    )(x2, eo[NPADE:], g[:S], g[S:], v1, v2)

    return y.reshape(1, S, H), aux[0, 0]


# speedup vs baseline: 1.4497x; 1.4497x over previous
import jax
import jax.numpy as jnp
import numpy as np
from jax.experimental import pallas as pl
from jax.experimental.pallas import tpu as pltpu

S, H = 2048, 1024
NH, NKV, HD = 16, 8, 64
I, E = 2048, 8
EPS = 1e-6
EP = 16
NA = 2 * S
BT = 256
NBE = NA // BT + E
NBS = S // BT
NBLK = NBE + NBS
NPADE = NBE * BT
NPAD = NBLK * BT
HW = H // 2


def _rms_mm_kernel(x_ref, ln_ref, w_ref, o_ref):
    x = x_ref[...]
    h = x * jax.lax.rsqrt(jnp.mean(x * x, axis=-1, keepdims=True) + EPS)
    h = h * ln_ref[...]
    o_ref[...] = jnp.dot(h.astype(jnp.bfloat16), w_ref[...],
                         preferred_element_type=jnp.float32).astype(jnp.bfloat16)


BQ = 512
BK = 512
NC = S // BQ


def _flash_kernel(q_ref, k_ref, v_ref, wo_ref, x_ref, o_ref,
                  m_scr, l_scr, acc_scr, oall_scr, mask_scr):
    ci = pl.program_id(0)
    hp = pl.program_id(1)
    cj = pl.program_id(2)
    scale = 1.0 / np.sqrt(HD)
    B2 = 2 * BQ

    @pl.when((ci == 0) & (hp == 0) & (cj == 0))
    def _():
        rl = jax.lax.broadcasted_iota(jnp.int32, (B2, BK), 0) % BQ
        cl = jax.lax.broadcasted_iota(jnp.int32, (B2, BK), 1)
        mask_scr[...] = jnp.where(cl <= rl, 0.0, -1e30)

    @pl.when(cj == 0)
    def _():
        m_scr[...] = jnp.full((B2, 128), -1e30, jnp.float32)
        l_scr[...] = jnp.zeros((B2, 128), jnp.float32)
        acc_scr[...] = jnp.zeros((B2, HD), jnp.float32)

    @pl.when(cj <= ci)
    def _():
        q2 = jnp.concatenate([q_ref[:, :HD], q_ref[:, HD:]], axis=0)
        s = jax.lax.dot_general(q2, k_ref[:, :HD], (((1,), (1,)), ((), ())),
                                preferred_element_type=jnp.float32) * scale
        s = jax.lax.cond(cj == ci, lambda: s + mask_scr[...], lambda: s)
        m_old = m_scr[:, 0:1]
        m_new = jnp.maximum(m_old, jnp.max(s, axis=-1, keepdims=True))
        p = jnp.exp(s - m_new)
        corr = jnp.exp(m_old - m_new)
        l_new = l_scr[:, 0:1] * corr + jnp.sum(p, axis=-1, keepdims=True)
        m_scr[...] = jnp.broadcast_to(m_new, (B2, 128))
        l_scr[...] = jnp.broadcast_to(l_new, (B2, 128))
        pv = jax.lax.dot_general(p.astype(jnp.bfloat16), v_ref[:, :HD],
                                 (((1,), (0,)), ((), ())),
                                 preferred_element_type=jnp.float32)
        acc_scr[...] = acc_scr[...] * corr + pv

    @pl.when(cj == ci)
    def _():
        o_h = acc_scr[...] / l_scr[:, 0:1]
        o_pair = jnp.concatenate([o_h[:BQ], o_h[BQ:]], axis=1)
        oall_scr[:, pl.ds(hp * 2 * HD, 2 * HD)] = o_pair.astype(jnp.bfloat16)

    @pl.when((hp == NKV - 1) & (cj == ci))
    def _():
        o_ref[...] = x_ref[...] + jax.lax.dot_general(
            oall_scr[...], wo_ref[...], (((1,), (0,)), ((), ())),
            preferred_element_type=jnp.float32)


def _router_kernel(x_ref, ln_ref, gw_ref, h_ref, p_ref, v1_ref, v2_ref,
                   meta_ref, aux_ref):
    x = x_ref[...]
    h = x * jax.lax.rsqrt(jnp.mean(x * x, axis=-1, keepdims=True) + EPS)
    h = h * ln_ref[...]
    h_ref[...] = h
    logits = jax.lax.dot_general(h, gw_ref[...], (((1,), (0,)), ((), ())),
                                 precision=jax.lax.Precision.HIGHEST,
                                 preferred_element_type=jnp.float32)
    lane = jax.lax.broadcasted_iota(jnp.int32, (S, EP), 1)
    logits = jnp.where(lane < E, logits, -jnp.inf)
    lm = jnp.max(logits, axis=-1, keepdims=True)
    ex = jnp.exp(logits - lm)
    probs = ex / jnp.sum(ex, axis=-1, keepdims=True)
    m1 = jnp.max(probs, axis=-1, keepdims=True)
    i1 = jnp.min(jnp.where(probs == m1, lane, EP), axis=-1, keepdims=True)
    mask1 = lane == i1
    pr2 = jnp.where(mask1, -jnp.inf, probs)
    m2 = jnp.max(pr2, axis=-1, keepdims=True)
    i2 = jnp.min(jnp.where(pr2 == m2, lane, EP), axis=-1, keepdims=True)
    mask2 = lane == i2
    v1_ref[...] = m1
    v2_ref[...] = m2
    counts = jnp.sum(jnp.where(mask1 | mask2, 1.0, 0.0), axis=0, keepdims=True)
    importance = jnp.mean(probs, axis=0, keepdims=True)
    aux = jnp.sum(importance * counts) * (E / (NA * 1.0))
    aux_ref[...] = jnp.full((1, 128), aux, jnp.float32)
    O = jnp.concatenate([jnp.where(mask1, 1.0, 0.0),
                         jnp.where(mask2, 1.0, 0.0)], axis=0)
    CB = 512
    rl = jax.lax.broadcasted_iota(jnp.int32, (CB, CB), 0)
    cl = jax.lax.broadcasted_iota(jnp.int32, (CB, CB), 1)
    ltri = jnp.where(cl < rl, 1.0, 0.0)
    carry = jnp.zeros((1, EP), jnp.float32)
    ranks = []
    for i in range(NA // CB):
        ob = O[i * CB:(i + 1) * CB]
        cb = jax.lax.dot_general(ltri, ob, (((1,), (0,)), ((), ())),
                                 preferred_element_type=jnp.float32) + carry
        ranks.append(jnp.sum(cb * ob, axis=-1, keepdims=True))
        carry = carry + jnp.sum(ob, axis=0, keepdims=True)
    rank = jnp.concatenate(ranks, axis=0)
    nb_row = jnp.floor((carry + (BT - 1.0)) * (1.0 / BT))
    er = jax.lax.broadcasted_iota(jnp.int32, (EP, EP), 0)
    ec = jax.lax.broadcasted_iota(jnp.int32, (EP, EP), 1)
    ustrict = jnp.where(er < ec, 1.0, 0.0)
    pad_off = jax.lax.dot_general(nb_row, ustrict, (((1,), (0,)), ((), ())),
                                  preferred_element_type=jnp.float32) * BT
    off = jnp.sum(O * pad_off, axis=-1, keepdims=True)
    p_ref[...] = (rank + off).astype(jnp.int32)
    ones_col = jnp.ones((NA, 1), jnp.float32)
    counts_col = jax.lax.dot_general(O, ones_col, (((0,), (0,)), ((), ())),
                                     preferred_element_type=jnp.float32)
    nb_col = jnp.floor((counts_col + (BT - 1.0)) * (1.0 / BT))
    lincl = jnp.where(er >= ec, 1.0, 0.0)
    cum_col = jax.lax.dot_general(lincl, nb_col, (((1,), (0,)), ((), ())),
                                  preferred_element_type=jnp.float32)
    e_col = jax.lax.broadcasted_iota(jnp.int32, (EP, 1), 0)
    b_iota = jax.lax.broadcasted_iota(jnp.int32, (1, 64), 1)
    mat = jnp.where((b_iota >= cum_col.astype(jnp.int32)) & (e_col < E),
                    1.0, 0.0)
    eob = jnp.sum(mat, axis=0, keepdims=True)
    nused = jnp.sum(jnp.where(e_col < E, nb_col, 0.0))
    meta_ref[...] = jnp.where(b_iota == 32, nused, eob).astype(jnp.int32)


def _scatter_kernel(p_ref, h_ref, xs_ref):
    xs_ref[NPADE:, :] = h_ref[...]

    def body(t, _):
        row = h_ref[pl.ds(t, 1), :]
        xs_ref[pl.ds(p_ref[t], 1), :] = row
        xs_ref[pl.ds(p_ref[S + t], 1), :] = row
        return 0

    jax.lax.fori_loop(0, S, body, 0)


def _gmm_kernel(eob_ref, nused_ref, xs_ref, w1_ref, w2_ref, o_ref):
    b = pl.program_id(0)

    @pl.when((b < nused_ref[0]) | (b >= NBE))
    def _():
        h1 = jax.lax.dot_general(xs_ref[...].astype(jnp.bfloat16), w1_ref[0],
                                 (((1,), (1,)), ((), ())),
                                 preferred_element_type=jnp.float32)
        h1 = (h1 * jax.nn.sigmoid(h1)).astype(jnp.bfloat16)
        o_ref[...] = jax.lax.dot_general(
            h1, w2_ref[0], (((1,), (1,)), ((), ())),
            preferred_element_type=jnp.float32)


def _combine_kernel(p_ref, x_ref, eo_ref, v1_ref, v2_ref, o_ref):
    ci = pl.program_id(0)

    def body(j, _):
        t = ci * BQ + j
        es = eo_ref[pl.ds(NPADE + t, 1), :]
        g1 = eo_ref[pl.ds(p_ref[t], 1), :]
        g2 = eo_ref[pl.ds(p_ref[S + t], 1), :]
        v1 = v1_ref[pl.ds(j, 1), :]
        v2 = v2_ref[pl.ds(j, 1), :]
        o_ref[pl.ds(j, 1), :] = (x_ref[pl.ds(j, 1), :] + es
                                 + v1 * g1 + v2 * g2)
        return 0

    jax.lax.fori_loop(0, BQ, body, 0)


def kernel(x, wq, wk, wv, wo, gate_w, w1, w2, sw1, sw2, ln1, ln2):
    xf = x[0]
    wk_p = jnp.pad(wk.T.reshape(H, NKV, HD), ((0, 0), (0, 0), (0, HD)))
    wv_p = jnp.pad(wv.T.reshape(H, NKV, HD), ((0, 0), (0, 0), (0, HD)))
    wqkv_t = jnp.concatenate(
        [wq.T, wk_p.reshape(H, 2 * NKV * HD), wv_p.reshape(H, 2 * NKV * HD)],
        axis=1).astype(jnp.bfloat16)
    wo_t = wo.T.astype(jnp.bfloat16)
    gw_t = jnp.pad(gate_w, ((0, EP - E), (0, 0))).T
    w1s = jnp.concatenate([w1, sw1[None]], axis=0).astype(jnp.bfloat16)
    w2s = jnp.concatenate([w2, sw2[None]], axis=0).astype(jnp.bfloat16)
    QW = NH * HD + 4 * NKV * HD

    qkv = pl.pallas_call(
        _rms_mm_kernel,
        out_shape=jax.ShapeDtypeStruct((S, QW), jnp.bfloat16),
    )(xf, ln1.reshape(1, H), wqkv_t)

    x2 = pl.pallas_call(
        _flash_kernel,
        grid=(NC, NKV, S // BK),
        in_specs=[
            pl.BlockSpec((BQ, 128), lambda ci, hp, cj: (ci, hp)),
            pl.BlockSpec((BK, 128), lambda ci, hp, cj: (cj, NKV + hp)),
            pl.BlockSpec((BK, 128), lambda ci, hp, cj: (cj, 2 * NKV + hp)),
            pl.BlockSpec((NH * HD, H), lambda ci, hp, cj: (0, 0)),
            pl.BlockSpec((BQ, H), lambda ci, hp, cj: (ci, 0)),
        ],
        out_specs=pl.BlockSpec((BQ, H), lambda ci, hp, cj: (ci, 0)),
        out_shape=jax.ShapeDtypeStruct((S, H), jnp.float32),
        scratch_shapes=[
            pltpu.VMEM((2 * BQ, 128), jnp.float32),
            pltpu.VMEM((2 * BQ, 128), jnp.float32),
            pltpu.VMEM((2 * BQ, HD), jnp.float32),
            pltpu.VMEM((BQ, NH * HD), jnp.bfloat16),
            pltpu.VMEM((2 * BQ, BK), jnp.float32),
        ],
        compiler_params=pltpu.CompilerParams(
            dimension_semantics=("arbitrary", "arbitrary", "arbitrary")),
    )(qkv, qkv, qkv, wo_t, xf)

    h2, p_pos, v1, v2, meta, aux = pl.pallas_call(
        _router_kernel,
        out_shape=[
            jax.ShapeDtypeStruct((S, H), jnp.float32),
            jax.ShapeDtypeStruct((NA, 1), jnp.int32),
            jax.ShapeDtypeStruct((S, 1), jnp.float32),
            jax.ShapeDtypeStruct((S, 1), jnp.float32),
            jax.ShapeDtypeStruct((1, 64), jnp.int32),
            jax.ShapeDtypeStruct((1, 128), jnp.float32),
        ],
    )(x2, ln2.reshape(1, H), gw_t)

    p_flat = p_pos.reshape(NA)
    xs = pl.pallas_call(
        _scatter_kernel,
        grid_spec=pltpu.PrefetchScalarGridSpec(
            num_scalar_prefetch=1,
            grid=(1,),
            in_specs=[pl.BlockSpec((S, H), lambda i, p: (0, 0))],
            out_specs=pl.BlockSpec((NPAD, H), lambda i, p: (0, 0)),
        ),
        out_shape=jax.ShapeDtypeStruct((NPAD, H), jnp.float32),
    )(p_flat, h2)

    eob = meta[0, :32]
    nused = meta[0, 32:33]
    eo = pl.pallas_call(
        _gmm_kernel,
        grid_spec=pltpu.PrefetchScalarGridSpec(
            num_scalar_prefetch=2,
            grid=(NBLK,),
            in_specs=[
                pl.BlockSpec((BT, H), lambda b, eob, nu: (b, 0)),
                pl.BlockSpec((1, I, H), lambda b, eob, nu: (eob[b], 0, 0)),
                pl.BlockSpec((1, H, I), lambda b, eob, nu: (eob[b], 0, 0)),
            ],
            out_specs=pl.BlockSpec((BT, H), lambda b, eob, nu: (b, 0)),
        ),
        out_shape=jax.ShapeDtypeStruct((NPAD, H), jnp.float32),
        compiler_params=pltpu.CompilerParams(
            dimension_semantics=("arbitrary",)),
    )(eob, nused, xs, w1s, w2s)

    y = pl.pallas_call(
        _combine_kernel,
        grid_spec=pltpu.PrefetchScalarGridSpec(
            num_scalar_prefetch=1,
            grid=(NC,),
            in_specs=[
                pl.BlockSpec((BQ, H), lambda i, p: (i, 0)),
                pl.BlockSpec((NPAD, H), lambda i, p: (0, 0)),
                pl.BlockSpec((BQ, 1), lambda i, p: (i, 0)),
                pl.BlockSpec((BQ, 1), lambda i, p: (i, 0)),
            ],
            out_specs=pl.BlockSpec((BQ, H), lambda i, p: (i, 0)),
        ),
        out_shape=jax.ShapeDtypeStruct((S, H), jnp.float32),
    )(p_flat, x2, eo, v1, v2)

    return y.reshape(1, S, H), aux[0, 0]


# speedup vs baseline: 1.7971x; 1.2397x over previous
import jax
import jax.numpy as jnp
import numpy as np
from jax.experimental import pallas as pl
from jax.experimental.pallas import tpu as pltpu

S, H = 2048, 1024
NH, NKV, HD = 16, 8, 64
I, E = 2048, 8
EPS = 1e-6
EP = 16
NA = 2 * S
BT = 256
NBE = NA // BT + E
NBS = S // BT
NBLK = NBE + NBS
NPADE = NBE * BT
NPAD = NBLK * BT
HW = H // 2


def _rms_mm_kernel(x_ref, ln_ref, w_ref, o_ref):
    x = x_ref[...]
    h = x * jax.lax.rsqrt(jnp.mean(x * x, axis=-1, keepdims=True) + EPS)
    h = h * ln_ref[...]
    o_ref[...] = jnp.dot(h.astype(jnp.bfloat16), w_ref[...],
                         preferred_element_type=jnp.float32).astype(jnp.bfloat16)


BQ = 512
BK = 1024
NC = S // BQ
NJ = S // BK


def _flash_kernel(q_ref, k_ref, v_ref, wo_ref, x_ref, o_ref,
                  l_scr, acc_scr, oall_scr, me_scr, mo_scr):
    ci = pl.program_id(0)
    hp = pl.program_id(1)
    cj = pl.program_id(2)
    scale = 1.0 / np.sqrt(HD)
    B2 = 2 * BQ

    @pl.when((ci == 0) & (hp == 0) & (cj == 0))
    def _():
        rl = jax.lax.broadcasted_iota(jnp.int32, (B2, BK), 0) % BQ
        cl = jax.lax.broadcasted_iota(jnp.int32, (B2, BK), 1)
        me_scr[...] = jnp.where(cl <= rl, 0.0, -1e30)
        mo_scr[...] = jnp.where(cl <= BQ + rl, 0.0, -1e30)

    @pl.when(cj == 0)
    def _():
        l_scr[...] = jnp.zeros((B2, 128), jnp.float32)
        acc_scr[...] = jnp.zeros((B2, HD), jnp.float32)

    @pl.when(cj <= ci // 2)
    def _():
        q2 = jnp.concatenate([q_ref[:, :HD], q_ref[:, HD:]], axis=0)
        s = jax.lax.dot_general(q2, k_ref[:, :HD], (((1,), (1,)), ((), ())),
                                preferred_element_type=jnp.float32) * scale
        s = jax.lax.cond(
            cj == ci // 2,
            lambda: jax.lax.cond(ci % 2 == 0,
                                 lambda: s + me_scr[...],
                                 lambda: s + mo_scr[...]),
            lambda: s)
        p = jnp.exp(s)
        l_scr[:, 0:1] += jnp.sum(p, axis=-1, keepdims=True)
        acc_scr[...] += jax.lax.dot_general(
            p.astype(jnp.bfloat16), v_ref[:, :HD], (((1,), (0,)), ((), ())),
            preferred_element_type=jnp.float32)

    @pl.when(cj == ci // 2)
    def _():
        o_h = acc_scr[...] / l_scr[:, 0:1]
        o_pair = jnp.concatenate([o_h[:BQ], o_h[BQ:]], axis=1)
        oall_scr[:, pl.ds(hp * 2 * HD, 2 * HD)] = o_pair.astype(jnp.bfloat16)

    @pl.when((hp == NKV - 1) & (cj == ci // 2))
    def _():
        o_ref[...] = x_ref[...] + jax.lax.dot_general(
            oall_scr[...], wo_ref[...], (((1,), (0,)), ((), ())),
            preferred_element_type=jnp.float32)


def _router_kernel(x_ref, ln_ref, gw_ref, h_ref, p_ref, v1_ref, v2_ref,
                   meta_ref, aux_ref):
    x = x_ref[...]
    h = x * jax.lax.rsqrt(jnp.mean(x * x, axis=-1, keepdims=True) + EPS)
    h = h * ln_ref[...]
    h_ref[...] = h
    logits = jax.lax.dot_general(h, gw_ref[...], (((1,), (0,)), ((), ())),
                                 precision=jax.lax.Precision.HIGHEST,
                                 preferred_element_type=jnp.float32)
    lane = jax.lax.broadcasted_iota(jnp.int32, (S, EP), 1)
    logits = jnp.where(lane < E, logits, -jnp.inf)
    lm = jnp.max(logits, axis=-1, keepdims=True)
    ex = jnp.exp(logits - lm)
    probs = ex / jnp.sum(ex, axis=-1, keepdims=True)
    m1 = jnp.max(probs, axis=-1, keepdims=True)
    i1 = jnp.min(jnp.where(probs == m1, lane, EP), axis=-1, keepdims=True)
    mask1 = lane == i1
    pr2 = jnp.where(mask1, -jnp.inf, probs)
    m2 = jnp.max(pr2, axis=-1, keepdims=True)
    i2 = jnp.min(jnp.where(pr2 == m2, lane, EP), axis=-1, keepdims=True)
    mask2 = lane == i2
    v1_ref[...] = m1
    v2_ref[...] = m2
    counts = jnp.sum(jnp.where(mask1 | mask2, 1.0, 0.0), axis=0, keepdims=True)
    importance = jnp.mean(probs, axis=0, keepdims=True)
    aux = jnp.sum(importance * counts) * (E / (NA * 1.0))
    aux_ref[...] = jnp.full((1, 128), aux, jnp.float32)
    O = jnp.concatenate([jnp.where(mask1, 1.0, 0.0),
                         jnp.where(mask2, 1.0, 0.0)], axis=0)
    CB = 512
    rl = jax.lax.broadcasted_iota(jnp.int32, (CB, CB), 0)
    cl = jax.lax.broadcasted_iota(jnp.int32, (CB, CB), 1)
    ltri = jnp.where(cl < rl, 1.0, 0.0).astype(jnp.bfloat16)
    Ob16 = O.astype(jnp.bfloat16)
    carry = jnp.zeros((1, EP), jnp.float32)
    ranks = []
    for i in range(NA // CB):
        ob = Ob16[i * CB:(i + 1) * CB]
        cb = jax.lax.dot_general(ltri, ob, (((1,), (0,)), ((), ())),
                                 preferred_element_type=jnp.float32) + carry
        ranks.append(jnp.sum(cb * O[i * CB:(i + 1) * CB],
                             axis=-1, keepdims=True))
        carry = carry + jnp.sum(O[i * CB:(i + 1) * CB], axis=0, keepdims=True)
    rank = jnp.concatenate(ranks, axis=0)
    nb_row = jnp.floor((carry + (BT - 1.0)) * (1.0 / BT))
    er = jax.lax.broadcasted_iota(jnp.int32, (EP, EP), 0)
    ec = jax.lax.broadcasted_iota(jnp.int32, (EP, EP), 1)
    ustrict = jnp.where(er < ec, 1.0, 0.0)
    pad_off = jax.lax.dot_general(nb_row, ustrict, (((1,), (0,)), ((), ())),
                                  preferred_element_type=jnp.float32) * BT
    off = jnp.sum(O * pad_off, axis=-1, keepdims=True)
    p_ref[...] = (rank + off).astype(jnp.int32)
    ones_col = jnp.ones((NA, 1), jnp.float32)
    counts_col = jax.lax.dot_general(O, ones_col, (((0,), (0,)), ((), ())),
                                     preferred_element_type=jnp.float32)
    nb_col = jnp.floor((counts_col + (BT - 1.0)) * (1.0 / BT))
    lincl = jnp.where(er >= ec, 1.0, 0.0)
    cum_col = jax.lax.dot_general(lincl, nb_col, (((1,), (0,)), ((), ())),
                                  preferred_element_type=jnp.float32)
    e_col = jax.lax.broadcasted_iota(jnp.int32, (EP, 1), 0)
    b_iota = jax.lax.broadcasted_iota(jnp.int32, (1, 64), 1)
    mat = jnp.where((b_iota >= cum_col.astype(jnp.int32)) & (e_col < E),
                    1.0, 0.0)
    eob = jnp.sum(mat, axis=0, keepdims=True)
    nused = jnp.sum(jnp.where(e_col < E, nb_col, 0.0))
    meta_ref[...] = jnp.where(b_iota == 32, nused, eob).astype(jnp.int32)


def _scatter_kernel(p_ref, h_ref, xs_ref):
    xs_ref[NPADE:, :] = h_ref[...]

    def body(t, _):
        row = h_ref[pl.ds(t, 1), :]
        xs_ref[pl.ds(p_ref[t], 1), :] = row
        xs_ref[pl.ds(p_ref[S + t], 1), :] = row
        return 0

    jax.lax.fori_loop(0, S, body, 0, unroll=8)


def _gmm_kernel(eob_ref, nused_ref, xs_ref, w1_ref, w2_ref, o_ref):
    b = pl.program_id(0)
    active = (b < nused_ref[0]) | (b >= NBE)

    @pl.when(active)
    def _():
        xs = xs_ref[...]
        xs = jnp.where(jnp.abs(xs) < 1e30, xs, 0.0)
        h1 = jax.lax.dot_general(xs.astype(jnp.bfloat16), w1_ref[0],
                                 (((1,), (1,)), ((), ())),
                                 preferred_element_type=jnp.float32)
        h1 = (h1 * jax.nn.sigmoid(h1)).astype(jnp.bfloat16)
        o_ref[...] = jax.lax.dot_general(
            h1, w2_ref[0], (((1,), (1,)), ((), ())),
            preferred_element_type=jnp.float32).astype(jnp.bfloat16)

    @pl.when(jnp.logical_not(active))
    def _():
        o_ref[...] = jnp.zeros((BT, H), jnp.bfloat16)


PB = 1024


def _combine_kernel(p1_ref, p2_ref, v1_ref, v2_ref, x_ref, eo_ref, o_ref):
    cj = pl.program_id(0)
    ci = pl.program_id(1)

    @pl.when((cj == 0) & (ci == 0))
    def _():
        o_ref[...] = x_ref[...]

    pos = jax.lax.broadcasted_iota(jnp.int32, (BQ, PB), 1) + cj * PB
    trow = jax.lax.broadcasted_iota(jnp.int32, (BQ, PB), 0) + ci * BQ
    w = (jnp.where(pos == p1_ref[...], v1_ref[...], 0.0)
         + jnp.where(pos == p2_ref[...], v2_ref[...], 0.0)
         + jnp.where(pos == NPADE + trow, 1.0, 0.0))
    o_ref[pl.ds(ci * BQ, BQ), :] += jax.lax.dot_general(
        w.astype(jnp.bfloat16), eo_ref[...], (((1,), (0,)), ((), ())),
        preferred_element_type=jnp.float32)


def kernel(x, wq, wk, wv, wo, gate_w, w1, w2, sw1, sw2, ln1, ln2):
    xf = x[0]
    wk_p = jnp.pad(wk.T.reshape(H, NKV, HD), ((0, 0), (0, 0), (0, HD)))
    wv_p = jnp.pad(wv.T.reshape(H, NKV, HD), ((0, 0), (0, 0), (0, HD)))
    wqkv_t = jnp.concatenate(
        [wq.T, wk_p.reshape(H, 2 * NKV * HD), wv_p.reshape(H, 2 * NKV * HD)],
        axis=1).astype(jnp.bfloat16)
    wo_t = wo.T.astype(jnp.bfloat16)
    gw_t = jnp.pad(gate_w, ((0, EP - E), (0, 0))).T
    w1s = jnp.concatenate([w1, sw1[None]], axis=0).astype(jnp.bfloat16)
    w2s = jnp.concatenate([w2, sw2[None]], axis=0).astype(jnp.bfloat16)
    QW = NH * HD + 4 * NKV * HD

    qkv = pl.pallas_call(
        _rms_mm_kernel,
        out_shape=jax.ShapeDtypeStruct((S, QW), jnp.bfloat16),
    )(xf, ln1.reshape(1, H), wqkv_t)

    x2 = pl.pallas_call(
        _flash_kernel,
        grid=(NC, NKV, NJ),
        in_specs=[
            pl.BlockSpec((BQ, 128), lambda ci, hp, cj: (ci, hp)),
            pl.BlockSpec((BK, 128), lambda ci, hp, cj: (cj, NKV + hp)),
            pl.BlockSpec((BK, 128), lambda ci, hp, cj: (cj, 2 * NKV + hp)),
            pl.BlockSpec((NH * HD, H), lambda ci, hp, cj: (0, 0)),
            pl.BlockSpec((BQ, H), lambda ci, hp, cj: (ci, 0)),
        ],
        out_specs=pl.BlockSpec((BQ, H), lambda ci, hp, cj: (ci, 0)),
        out_shape=jax.ShapeDtypeStruct((S, H), jnp.float32),
        scratch_shapes=[
            pltpu.VMEM((2 * BQ, 128), jnp.float32),
            pltpu.VMEM((2 * BQ, HD), jnp.float32),
            pltpu.VMEM((BQ, NH * HD), jnp.bfloat16),
            pltpu.VMEM((2 * BQ, BK), jnp.float32),
            pltpu.VMEM((2 * BQ, BK), jnp.float32),
        ],
        compiler_params=pltpu.CompilerParams(
            dimension_semantics=("arbitrary", "arbitrary", "arbitrary")),
    )(qkv, qkv, qkv, wo_t, xf)

    h2, p_pos, v1, v2, meta, aux = pl.pallas_call(
        _router_kernel,
        out_shape=[
            jax.ShapeDtypeStruct((S, H), jnp.float32),
            jax.ShapeDtypeStruct((NA, 1), jnp.int32),
            jax.ShapeDtypeStruct((S, 1), jnp.float32),
            jax.ShapeDtypeStruct((S, 1), jnp.float32),
            jax.ShapeDtypeStruct((1, 64), jnp.int32),
            jax.ShapeDtypeStruct((1, 128), jnp.float32),
        ],
    )(x2, ln2.reshape(1, H), gw_t)

    p_flat = p_pos.reshape(NA)
    xs = pl.pallas_call(
        _scatter_kernel,
        grid_spec=pltpu.PrefetchScalarGridSpec(
            num_scalar_prefetch=1,
            grid=(1,),
            in_specs=[pl.BlockSpec((S, H), lambda i, p: (0, 0))],
            out_specs=pl.BlockSpec((NPAD, H), lambda i, p: (0, 0)),
        ),
        out_shape=jax.ShapeDtypeStruct((NPAD, H), jnp.float32),
    )(p_flat, h2)

    eob = meta[0, :32]
    nused = meta[0, 32:33]
    eo = pl.pallas_call(
        _gmm_kernel,
        grid_spec=pltpu.PrefetchScalarGridSpec(
            num_scalar_prefetch=2,
            grid=(NBLK,),
            in_specs=[
                pl.BlockSpec((BT, H), lambda b, eob, nu: (b, 0)),
                pl.BlockSpec((1, I, H), lambda b, eob, nu: (eob[b], 0, 0)),
                pl.BlockSpec((1, H, I), lambda b, eob, nu: (eob[b], 0, 0)),
            ],
            out_specs=pl.BlockSpec((BT, H), lambda b, eob, nu: (b, 0)),
        ),
        out_shape=jax.ShapeDtypeStruct((NPAD, H), jnp.bfloat16),
        compiler_params=pltpu.CompilerParams(
            dimension_semantics=("arbitrary",)),
    )(eob, nused, xs, w1s, w2s)

    y = pl.pallas_call(
        _combine_kernel,
        grid=(NPAD // PB, NC),
        in_specs=[
            pl.BlockSpec((BQ, 1), lambda cj, ci: (ci, 0)),
            pl.BlockSpec((BQ, 1), lambda cj, ci: (ci, 0)),
            pl.BlockSpec((BQ, 1), lambda cj, ci: (ci, 0)),
            pl.BlockSpec((BQ, 1), lambda cj, ci: (ci, 0)),
            pl.BlockSpec((S, H), lambda cj, ci: (0, 0)),
            pl.BlockSpec((PB, H), lambda cj, ci: (cj, 0)),
        ],
        out_specs=pl.BlockSpec((S, H), lambda cj, ci: (0, 0)),
        out_shape=jax.ShapeDtypeStruct((S, H), jnp.float32),
        compiler_params=pltpu.CompilerParams(
            dimension_semantics=("arbitrary", "arbitrary")),
    )(p_pos[:S], p_pos[S:], v1, v2, x2, eo)

    return y.reshape(1, S, H), aux[0, 0]


# speedup vs baseline: 1.8267x; 1.0165x over previous
import jax
import jax.numpy as jnp
import numpy as np
from jax.experimental import pallas as pl
from jax.experimental.pallas import tpu as pltpu

S, H = 2048, 1024
NH, NKV, HD = 16, 8, 64
I, E = 2048, 8
EPS = 1e-6
EP = 16
NA = 2 * S
BT = 256
NBE = NA // BT + E
NBS = S // BT
NBLK = NBE + NBS
NPADE = NBE * BT
NPAD = NBLK * BT
HW = H // 2


def _rms_mm_kernel(x_ref, ln_ref, w_ref, o_ref):
    x = x_ref[...]
    h = x * jax.lax.rsqrt(jnp.mean(x * x, axis=-1, keepdims=True) + EPS)
    h = h * ln_ref[...]
    o_ref[...] = jnp.dot(h.astype(jnp.bfloat16), w_ref[...],
                         preferred_element_type=jnp.float32).astype(jnp.bfloat16)


BQ = 512
BK = 1024
NC = S // BQ
NJ = S // BK


def _flash_kernel(q_ref, k_ref, v_ref, wo_ref, x_ref, o_ref,
                  acc_scr, oall_scr, me_scr, mo_scr, onec_scr):
    ci = pl.program_id(0)
    hp = pl.program_id(1)
    cj = pl.program_id(2)
    B2 = 2 * BQ

    @pl.when((ci == 0) & (hp == 0) & (cj == 0))
    def _():
        rl = jax.lax.broadcasted_iota(jnp.int32, (B2, BK), 0) % BQ
        cl = jax.lax.broadcasted_iota(jnp.int32, (B2, BK), 1)
        me_scr[...] = jnp.where(cl <= rl, 0.0, -1e30)
        mo_scr[...] = jnp.where(cl <= BQ + rl, 0.0, -1e30)
        lane = jax.lax.broadcasted_iota(jnp.int32, (BK, 128), 1)
        onec_scr[...] = jnp.where(lane == HD, 1.0, 0.0).astype(jnp.bfloat16)

    @pl.when(cj == 0)
    def _():
        acc_scr[...] = jnp.zeros((B2, 128), jnp.float32)

    @pl.when(cj <= ci // 2)
    def _():
        q2 = jnp.concatenate([q_ref[:, :HD], q_ref[:, HD:]], axis=0)
        s = jax.lax.dot_general(q2, k_ref[:, :HD], (((1,), (1,)), ((), ())),
                                preferred_element_type=jnp.float32)
        s = jax.lax.cond(
            cj == ci // 2,
            lambda: jax.lax.cond(ci % 2 == 0,
                                 lambda: s + me_scr[...],
                                 lambda: s + mo_scr[...]),
            lambda: s)
        p = jnp.exp(s).astype(jnp.bfloat16)
        vext = v_ref[...] + onec_scr[...]
        acc_scr[...] += jax.lax.dot_general(
            p, vext, (((1,), (0,)), ((), ())),
            preferred_element_type=jnp.float32)

    @pl.when(cj == ci // 2)
    def _():
        o_h = acc_scr[:, :HD] / acc_scr[:, HD:HD + 1]
        o_pair = jnp.concatenate([o_h[:BQ], o_h[BQ:]], axis=1)
        oall_scr[:, pl.ds(hp * 2 * HD, 2 * HD)] = o_pair.astype(jnp.bfloat16)

    @pl.when((hp == NKV - 1) & (cj == ci // 2))
    def _():
        o_ref[...] = x_ref[...] + jax.lax.dot_general(
            oall_scr[...], wo_ref[...], (((1,), (0,)), ((), ())),
            preferred_element_type=jnp.float32)


def _router_kernel(x_ref, ln_ref, gw_ref, h_ref, p_ref, v1_ref, v2_ref,
                   meta_ref, aux_ref):
    x = x_ref[...]
    h = x * jax.lax.rsqrt(jnp.mean(x * x, axis=-1, keepdims=True) + EPS)
    h = h * ln_ref[...]
    h_ref[...] = h
    logits = jax.lax.dot_general(h, gw_ref[...], (((1,), (0,)), ((), ())),
                                 precision=jax.lax.Precision.HIGHEST,
                                 preferred_element_type=jnp.float32)
    lane = jax.lax.broadcasted_iota(jnp.int32, (S, EP), 1)
    logits = jnp.where(lane < E, logits, -jnp.inf)
    lm = jnp.max(logits, axis=-1, keepdims=True)
    ex = jnp.exp(logits - lm)
    probs = ex / jnp.sum(ex, axis=-1, keepdims=True)
    m1 = jnp.max(probs, axis=-1, keepdims=True)
    i1 = jnp.min(jnp.where(probs == m1, lane, EP), axis=-1, keepdims=True)
    mask1 = lane == i1
    pr2 = jnp.where(mask1, -jnp.inf, probs)
    m2 = jnp.max(pr2, axis=-1, keepdims=True)
    i2 = jnp.min(jnp.where(pr2 == m2, lane, EP), axis=-1, keepdims=True)
    mask2 = lane == i2
    v1_ref[...] = m1
    v2_ref[...] = m2
    counts = jnp.sum(jnp.where(mask1 | mask2, 1.0, 0.0), axis=0, keepdims=True)
    importance = jnp.mean(probs, axis=0, keepdims=True)
    aux = jnp.sum(importance * counts) * (E / (NA * 1.0))
    aux_ref[...] = jnp.full((1, 128), aux, jnp.float32)
    O = jnp.concatenate([jnp.where(mask1, 1.0, 0.0),
                         jnp.where(mask2, 1.0, 0.0)], axis=0)
    CB = 512
    rl = jax.lax.broadcasted_iota(jnp.int32, (CB, CB), 0)
    cl = jax.lax.broadcasted_iota(jnp.int32, (CB, CB), 1)
    ltri = jnp.where(cl < rl, 1.0, 0.0).astype(jnp.bfloat16)
    Ob16 = O.astype(jnp.bfloat16)
    carry = jnp.zeros((1, EP), jnp.float32)
    ranks = []
    for i in range(NA // CB):
        ob = Ob16[i * CB:(i + 1) * CB]
        cb = jax.lax.dot_general(ltri, ob, (((1,), (0,)), ((), ())),
                                 preferred_element_type=jnp.float32) + carry
        ranks.append(jnp.sum(cb * O[i * CB:(i + 1) * CB],
                             axis=-1, keepdims=True))
        carry = carry + jnp.sum(O[i * CB:(i + 1) * CB], axis=0, keepdims=True)
    rank = jnp.concatenate(ranks, axis=0)
    nb_row = jnp.floor((carry + (BT - 1.0)) * (1.0 / BT))
    er = jax.lax.broadcasted_iota(jnp.int32, (EP, EP), 0)
    ec = jax.lax.broadcasted_iota(jnp.int32, (EP, EP), 1)
    ustrict = jnp.where(er < ec, 1.0, 0.0)
    pad_off = jax.lax.dot_general(nb_row, ustrict, (((1,), (0,)), ((), ())),
                                  preferred_element_type=jnp.float32) * BT
    off = jnp.sum(O * pad_off, axis=-1, keepdims=True)
    p_ref[...] = (rank + off).astype(jnp.int32)
    ones_col = jnp.ones((NA, 1), jnp.float32)
    counts_col = jax.lax.dot_general(O, ones_col, (((0,), (0,)), ((), ())),
                                     preferred_element_type=jnp.float32)
    nb_col = jnp.floor((counts_col + (BT - 1.0)) * (1.0 / BT))
    lincl = jnp.where(er >= ec, 1.0, 0.0)
    cum_col = jax.lax.dot_general(lincl, nb_col, (((1,), (0,)), ((), ())),
                                  preferred_element_type=jnp.float32)
    e_col = jax.lax.broadcasted_iota(jnp.int32, (EP, 1), 0)
    b_iota = jax.lax.broadcasted_iota(jnp.int32, (1, 64), 1)
    mat = jnp.where((b_iota >= cum_col.astype(jnp.int32)) & (e_col < E),
                    1.0, 0.0)
    eob = jnp.sum(mat, axis=0, keepdims=True)
    nused = jnp.sum(jnp.where(e_col < E, nb_col, 0.0))
    meta_ref[...] = jnp.where(b_iota == 32, nused, eob).astype(jnp.int32)


def _scatter_kernel(p_ref, h_ref, xs_ref):
    xs_ref[NPADE:, :] = h_ref[...]

    def body(t, _):
        row = h_ref[pl.ds(t, 1), :]
        xs_ref[pl.ds(p_ref[t], 1), :] = row
        xs_ref[pl.ds(p_ref[S + t], 1), :] = row
        return 0

    jax.lax.fori_loop(0, S, body, 0, unroll=8)


def _gmm_kernel(eob_ref, nused_ref, xs_ref, w1_ref, w2_ref, o_ref):
    b = pl.program_id(0)
    active = (b < nused_ref[0]) | (b >= NBE)

    @pl.when(active)
    def _():
        xs = xs_ref[...]
        xs = jnp.where(jnp.abs(xs) < 1e30, xs, 0.0)
        h1 = jax.lax.dot_general(xs.astype(jnp.bfloat16), w1_ref[0],
                                 (((1,), (1,)), ((), ())),
                                 preferred_element_type=jnp.float32)
        h1 = (h1 * jax.nn.sigmoid(h1)).astype(jnp.bfloat16)
        o_ref[...] = jax.lax.dot_general(
            h1, w2_ref[0], (((1,), (1,)), ((), ())),
            preferred_element_type=jnp.float32).astype(jnp.bfloat16)

    @pl.when(jnp.logical_not(active))
    def _():
        o_ref[...] = jnp.zeros((BT, H), jnp.bfloat16)


PB = 1024


def _combine_kernel(p1_ref, p2_ref, v1_ref, v2_ref, x_ref, eo_ref, o_ref):
    cj = pl.program_id(0)
    ci = pl.program_id(1)

    @pl.when((cj == 0) & (ci == 0))
    def _():
        o_ref[...] = x_ref[...]

    pos = jax.lax.broadcasted_iota(jnp.int32, (BQ, PB), 1) + cj * PB
    trow = jax.lax.broadcasted_iota(jnp.int32, (BQ, PB), 0) + ci * BQ
    w = (jnp.where(pos == p1_ref[...], v1_ref[...], 0.0)
         + jnp.where(pos == p2_ref[...], v2_ref[...], 0.0)
         + jnp.where(pos == NPADE + trow, 1.0, 0.0))
    o_ref[pl.ds(ci * BQ, BQ), :] += jax.lax.dot_general(
        w.astype(jnp.bfloat16), eo_ref[...], (((1,), (0,)), ((), ())),
        preferred_element_type=jnp.float32)


def kernel(x, wq, wk, wv, wo, gate_w, w1, w2, sw1, sw2, ln1, ln2):
    xf = x[0]
    wk_p = jnp.pad(wk.T.reshape(H, NKV, HD), ((0, 0), (0, 0), (0, HD)))
    wv_p = jnp.pad(wv.T.reshape(H, NKV, HD), ((0, 0), (0, 0), (0, HD)))
    wqkv_t = jnp.concatenate(
        [wq.T * np.float32(1.0 / np.sqrt(HD)),
         wk_p.reshape(H, 2 * NKV * HD), wv_p.reshape(H, 2 * NKV * HD)],
        axis=1).astype(jnp.bfloat16)
    wo_t = wo.T.astype(jnp.bfloat16)
    gw_t = jnp.pad(gate_w, ((0, EP - E), (0, 0))).T
    w1s = jnp.concatenate([w1, sw1[None]], axis=0).astype(jnp.bfloat16)
    w2s = jnp.concatenate([w2, sw2[None]], axis=0).astype(jnp.bfloat16)
    QW = NH * HD + 4 * NKV * HD

    qkv = pl.pallas_call(
        _rms_mm_kernel,
        out_shape=jax.ShapeDtypeStruct((S, QW), jnp.bfloat16),
    )(xf, ln1.reshape(1, H), wqkv_t)

    x2 = pl.pallas_call(
        _flash_kernel,
        grid=(NC, NKV, NJ),
        in_specs=[
            pl.BlockSpec((BQ, 128), lambda ci, hp, cj: (ci, hp)),
            pl.BlockSpec((BK, 128), lambda ci, hp, cj: (cj, NKV + hp)),
            pl.BlockSpec((BK, 128), lambda ci, hp, cj: (cj, 2 * NKV + hp)),
            pl.BlockSpec((NH * HD, H), lambda ci, hp, cj: (0, 0)),
            pl.BlockSpec((BQ, H), lambda ci, hp, cj: (ci, 0)),
        ],
        out_specs=pl.BlockSpec((BQ, H), lambda ci, hp, cj: (ci, 0)),
        out_shape=jax.ShapeDtypeStruct((S, H), jnp.float32),
        scratch_shapes=[
            pltpu.VMEM((2 * BQ, 128), jnp.float32),
            pltpu.VMEM((BQ, NH * HD), jnp.bfloat16),
            pltpu.VMEM((2 * BQ, BK), jnp.float32),
            pltpu.VMEM((2 * BQ, BK), jnp.float32),
            pltpu.VMEM((BK, 128), jnp.bfloat16),
        ],
        compiler_params=pltpu.CompilerParams(
            dimension_semantics=("arbitrary", "arbitrary", "arbitrary")),
    )(qkv, qkv, qkv, wo_t, xf)

    h2, p_pos, v1, v2, meta, aux = pl.pallas_call(
        _router_kernel,
        out_shape=[
            jax.ShapeDtypeStruct((S, H), jnp.float32),
            jax.ShapeDtypeStruct((NA, 1), jnp.int32),
            jax.ShapeDtypeStruct((S, 1), jnp.float32),
            jax.ShapeDtypeStruct((S, 1), jnp.float32),
            jax.ShapeDtypeStruct((1, 64), jnp.int32),
            jax.ShapeDtypeStruct((1, 128), jnp.float32),
        ],
    )(x2, ln2.reshape(1, H), gw_t)

    p_flat = p_pos.reshape(NA)
    xs = pl.pallas_call(
        _scatter_kernel,
        grid_spec=pltpu.PrefetchScalarGridSpec(
            num_scalar_prefetch=1,
            grid=(1,),
            in_specs=[pl.BlockSpec((S, H), lambda i, p: (0, 0))],
            out_specs=pl.BlockSpec((NPAD, H), lambda i, p: (0, 0)),
        ),
        out_shape=jax.ShapeDtypeStruct((NPAD, H), jnp.float32),
    )(p_flat, h2)

    eob = meta[0, :32]
    nused = meta[0, 32:33]
    eo = pl.pallas_call(
        _gmm_kernel,
        grid_spec=pltpu.PrefetchScalarGridSpec(
            num_scalar_prefetch=2,
            grid=(NBLK,),
            in_specs=[
                pl.BlockSpec((BT, H), lambda b, eob, nu: (b, 0)),
                pl.BlockSpec((1, I, H), lambda b, eob, nu: (eob[b], 0, 0)),
                pl.BlockSpec((1, H, I), lambda b, eob, nu: (eob[b], 0, 0)),
            ],
            out_specs=pl.BlockSpec((BT, H), lambda b, eob, nu: (b, 0)),
        ),
        out_shape=jax.ShapeDtypeStruct((NPAD, H), jnp.bfloat16),
        compiler_params=pltpu.CompilerParams(
            dimension_semantics=("arbitrary",)),
    )(eob, nused, xs, w1s, w2s)

    y = pl.pallas_call(
        _combine_kernel,
        grid=(NPAD // PB, NC),
        in_specs=[
            pl.BlockSpec((BQ, 1), lambda cj, ci: (ci, 0)),
            pl.BlockSpec((BQ, 1), lambda cj, ci: (ci, 0)),
            pl.BlockSpec((BQ, 1), lambda cj, ci: (ci, 0)),
            pl.BlockSpec((BQ, 1), lambda cj, ci: (ci, 0)),
            pl.BlockSpec((S, H), lambda cj, ci: (0, 0)),
            pl.BlockSpec((PB, H), lambda cj, ci: (cj, 0)),
        ],
        out_specs=pl.BlockSpec((S, H), lambda cj, ci: (0, 0)),
        out_shape=jax.ShapeDtypeStruct((S, H), jnp.float32),
        compiler_params=pltpu.CompilerParams(
            dimension_semantics=("arbitrary", "arbitrary")),
    )(p_pos[:S], p_pos[S:], v1, v2, x2, eo)

    return y.reshape(1, S, H), aux[0, 0]
